# baseline (speedup 1.0000x reference)
import jax
import jax.numpy as jnp
from jax import lax
from jax.experimental import pallas as pl
from jax.experimental.pallas import tpu as pltpu
from jax.experimental.pallas import tpu_sc as plsc

B, N, C = 8, 20000, 91
L = 16
W = 512
NCHUNK = 5
RW = NCHUNK * W
SC_N = 4 * RW
TC_N = N - SC_N
NB = 2048
assert SC_N % NB == 0


def _sc_body(lg_hbm, bx_hbm, ts_hbm, o_s, o_n, o_l, o_b,
             lg_v, bx_v, s_v, n_v, l_v, ob_v, ts_v,
             isem0, isem1, osem0, osem1):
    cid = lax.axis_index("c")
    sid = lax.axis_index("s")
    wid = sid * 2 + cid
    img = lax.shift_right_logical(wid, 2)
    nbase = (wid & 3) * RW
    lane = lax.iota(jnp.int32, L)
    isem = (isem0, isem1)
    osem = (osem0, osem1)

    pltpu.sync_copy(ts_hbm, ts_v)
    swv = plsc.load_gather(ts_v, [lane * 0 + (2 * img + 1)]).astype(jnp.float32)
    shv = plsc.load_gather(ts_v, [lane * 0 + 2 * img]).astype(jnp.float32)

    def in_copies(k, slot):
        n0 = nbase + k * W
        return (
            pltpu.make_async_copy(lg_hbm.at[:, img, pl.ds(n0, W)],
                                  lg_v.at[slot], isem[slot]),
            pltpu.make_async_copy(bx_hbm.at[img, :, pl.ds(n0, W)],
                                  bx_v.at[slot], isem[slot]),
        )

    def out_copies(k, slot):
        n0 = nbase + k * W
        return (
            pltpu.make_async_copy(s_v.at[slot], o_s.at[img, pl.ds(n0, W)],
                                  osem[slot]),
            pltpu.make_async_copy(n_v.at[slot], o_n.at[img, pl.ds(n0, W)],
                                  osem[slot]),
            pltpu.make_async_copy(l_v.at[slot], o_l.at[img, pl.ds(n0, W)],
                                  osem[slot]),
            pltpu.make_async_copy(ob_v.at[slot], o_b.at[img, :, pl.ds(n0, W)],
                                  osem[slot]),
        )

    def compute(slot):
        def grp(g, c2):
            base = g * L
            dsl = pl.ds(base, L)
            m = lg_v[slot, 0, dsl]
            lbl = jnp.zeros((L,), jnp.int32)
            for c in range(1, C, 2):
                va = lg_v[slot, c, dsl]
                vb = lg_v[slot, c + 1, dsl]
                lp = jnp.full((L,), c, jnp.int32) + (vb > va).astype(jnp.int32)
                mp = jnp.maximum(va, vb)
                gt = mp > m
                m = jnp.maximum(m, mp)
                lbl = jnp.where(gt, lp, lbl)
            sig = 1.0 / (1.0 + jnp.exp(-m))
            s_v[slot, dsl] = sig
            n_v[slot, dsl] = 1.0 - sig
            l_v[slot, dsl] = lbl

            cx = bx_v[slot, 0, dsl]
            cy = bx_v[slot, 1, dsl]
            hw = bx_v[slot, 2, dsl] * 0.5
            hh = bx_v[slot, 3, dsl] * 0.5
            ob_v[slot, 0, dsl] = (cx - hw) * swv
            ob_v[slot, 1, dsl] = (cy - hh) * shv
            ob_v[slot, 2, dsl] = (cx + hw) * swv
            ob_v[slot, 3, dsl] = (cy + hh) * shv
            return c2

        lax.fori_loop(0, W // L, grp, 0)

    for k in range(min(2, NCHUNK)):
        for d in in_copies(k, k):
            d.start()
    for k in range(NCHUNK):
        slot = k % 2
        for d in in_copies(k, slot):
            d.wait()
        if k >= 2:
            for d in out_copies(k - 2, slot):
                d.wait()
        compute(slot)
        for d in out_copies(k, slot):
            d.start()
        if k + 2 < NCHUNK:
            for d in in_copies(k + 2, slot):
                d.start()
    for k in range(max(0, NCHUNK - 2), NCHUNK):
        for d in out_copies(k, k % 2):
            d.wait()


def _tc_body(lg_ref, bx_ref, ts_ref, s_ref, n_ref, l_ref, ob_ref):
    x = lg_ref[...]
    m = jnp.max(x, axis=0)
    iota = lax.broadcasted_iota(jnp.int32, x.shape, 0)
    lbl = jnp.min(jnp.where(x == m[None], iota, C), axis=0)
    sig = 1.0 / (1.0 + jnp.exp(-m))
    s_ref[...] = sig
    n_ref[...] = 1.0 - sig
    l_ref[...] = lbl

    tsf = ts_ref[...].astype(jnp.float32)
    hsz = tsf[:, 0:1]
    wsz = tsf[:, 1:2]
    cx = bx_ref[:, 0, :]
    cy = bx_ref[:, 1, :]
    hw = bx_ref[:, 2, :] * 0.5
    hh = bx_ref[:, 3, :] * 0.5
    ob_ref[:, 0, :] = (cx - hw) * wsz
    ob_ref[:, 1, :] = (cy - hh) * hsz
    ob_ref[:, 2, :] = (cx + hw) * wsz
    ob_ref[:, 3, :] = (cy + hh) * hsz


def kernel(pred_logits, pred_boxes, target_sizes):
    lg = jnp.transpose(pred_logits, (2, 0, 1))
    bx = jnp.transpose(pred_boxes, (0, 2, 1))
    ts = target_sizes.reshape(2 * B)

    mesh = plsc.VectorSubcoreMesh(core_axis_name="c", subcore_axis_name="s")
    sc_out_type = [
        jax.ShapeDtypeStruct((B, SC_N), jnp.float32),
        jax.ShapeDtypeStruct((B, SC_N), jnp.float32),
        jax.ShapeDtypeStruct((B, SC_N), jnp.int32),
        jax.ShapeDtypeStruct((B, 4, SC_N), jnp.float32),
    ]
    sc_scratch = [
        pltpu.VMEM((2, C, W), jnp.float32),
        pltpu.VMEM((2, 4, W), jnp.float32),
        pltpu.VMEM((2, W), jnp.float32),
        pltpu.VMEM((2, W), jnp.float32),
        pltpu.VMEM((2, W), jnp.int32),
        pltpu.VMEM((2, 4, W), jnp.float32),
        pltpu.VMEM((2 * B,), jnp.int32),
        pltpu.SemaphoreType.DMA,
        pltpu.SemaphoreType.DMA,
        pltpu.SemaphoreType.DMA,
        pltpu.SemaphoreType.DMA,
    ]
    sc_f = pl.kernel(_sc_body, out_type=sc_out_type, mesh=mesh,
                     scratch_types=sc_scratch,
                     compiler_params=pltpu.CompilerParams(
                         needs_layout_passes=False))
    sc_s, sc_n, sc_l, sc_b = sc_f(lg, bx, ts)

    nblk = (TC_N + NB - 1) // NB
    off = SC_N // NB
    tc_f = pl.pallas_call(
        _tc_body,
        grid=(nblk,),
        in_specs=[
            pl.BlockSpec((C, B, NB), lambda j: (0, 0, off + j)),
            pl.BlockSpec((B, 4, NB), lambda j: (0, 0, off + j)),
            pl.BlockSpec((B, 2), lambda j: (0, 0)),
        ],
        out_specs=[
            pl.BlockSpec((B, NB), lambda j: (0, j)),
            pl.BlockSpec((B, NB), lambda j: (0, j)),
            pl.BlockSpec((B, NB), lambda j: (0, j)),
            pl.BlockSpec((B, 4, NB), lambda j: (0, 0, j)),
        ],
        out_shape=[
            jax.ShapeDtypeStruct((B, TC_N), jnp.float32),
            jax.ShapeDtypeStruct((B, TC_N), jnp.float32),
            jax.ShapeDtypeStruct((B, TC_N), jnp.int32),
            jax.ShapeDtypeStruct((B, 4, TC_N), jnp.float32),
        ],
    )
    tc_s, tc_n, tc_l, tc_b = tc_f(lg, bx, target_sizes)

    s = jnp.concatenate([sc_s, tc_s], axis=1)
    n = jnp.concatenate([sc_n, tc_n], axis=1)
    l = jnp.concatenate([sc_l, tc_l], axis=1)
    b4 = jnp.concatenate([sc_b, tc_b], axis=2)
    return s, n, l, jnp.transpose(b4, (0, 2, 1))


# speedup vs baseline: 1.0599x; 1.0599x over previous
import jax
import jax.numpy as jnp
from jax import lax
from jax.experimental import pallas as pl
from jax.experimental.pallas import tpu as pltpu
from jax.experimental.pallas import tpu_sc as plsc

B, N, C = 8, 20000, 91
L = 16
W = 512
NCHUNK = 4
RW = NCHUNK * W
SC_N = 4 * RW
TC_N = N - SC_N
NB = 2048
assert SC_N % NB == 0


def _sc_body(lg_hbm, bx_hbm, ts_hbm, o_s, o_n, o_l, o_b,
             lg_v, bx_v, s_v, n_v, l_v, ob_v, ts_v,
             isem0, isem1, osem0, osem1):
    cid = lax.axis_index("c")
    sid = lax.axis_index("s")
    wid = sid * 2 + cid
    img = lax.shift_right_logical(wid, 2)
    nbase = (wid & 3) * RW
    lane = lax.iota(jnp.int32, L)
    isem = (isem0, isem1)
    osem = (osem0, osem1)

    pltpu.sync_copy(ts_hbm, ts_v)
    swv = plsc.load_gather(ts_v, [lane * 0 + (2 * img + 1)]).astype(jnp.float32)
    shv = plsc.load_gather(ts_v, [lane * 0 + 2 * img]).astype(jnp.float32)

    def in_copies(k, slot):
        n0 = nbase + k * W
        return (
            pltpu.make_async_copy(lg_hbm.at[:, img, pl.ds(n0, W)],
                                  lg_v.at[slot], isem[slot]),
            pltpu.make_async_copy(bx_hbm.at[img, :, pl.ds(n0, W)],
                                  bx_v.at[slot], isem[slot]),
        )

    def out_copies(k, slot):
        n0 = nbase + k * W
        return (
            pltpu.make_async_copy(s_v.at[slot], o_s.at[img, pl.ds(n0, W)],
                                  osem[slot]),
            pltpu.make_async_copy(n_v.at[slot], o_n.at[img, pl.ds(n0, W)],
                                  osem[slot]),
            pltpu.make_async_copy(l_v.at[slot], o_l.at[img, pl.ds(n0, W)],
                                  osem[slot]),
            pltpu.make_async_copy(ob_v.at[slot], o_b.at[img, :, pl.ds(n0, W)],
                                  osem[slot]),
        )

    def compute(slot):
        def grp(g, c2):
            base = g * L
            dsl = pl.ds(base, L)
            m = lg_v[slot, 0, dsl]
            lbl = jnp.zeros((L,), jnp.int32)
            for c in range(1, C, 2):
                va = lg_v[slot, c, dsl]
                vb = lg_v[slot, c + 1, dsl]
                lp = jnp.full((L,), c, jnp.int32) + (vb > va).astype(jnp.int32)
                mp = jnp.maximum(va, vb)
                gt = mp > m
                m = jnp.maximum(m, mp)
                lbl = jnp.where(gt, lp, lbl)
            sig = 1.0 / (1.0 + jnp.exp(-m))
            s_v[slot, dsl] = sig
            n_v[slot, dsl] = 1.0 - sig
            l_v[slot, dsl] = lbl

            cx = bx_v[slot, 0, dsl]
            cy = bx_v[slot, 1, dsl]
            hw = bx_v[slot, 2, dsl] * 0.5
            hh = bx_v[slot, 3, dsl] * 0.5
            ob_v[slot, 0, dsl] = (cx - hw) * swv
            ob_v[slot, 1, dsl] = (cy - hh) * shv
            ob_v[slot, 2, dsl] = (cx + hw) * swv
            ob_v[slot, 3, dsl] = (cy + hh) * shv
            return c2

        lax.fori_loop(0, W // L, grp, 0)

    for k in range(min(2, NCHUNK)):
        for d in in_copies(k, k):
            d.start()
    for k in range(NCHUNK):
        slot = k % 2
        for d in in_copies(k, slot):
            d.wait()
        if k >= 2:
            for d in out_copies(k - 2, slot):
                d.wait()
        compute(slot)
        for d in out_copies(k, slot):
            d.start()
        if k + 2 < NCHUNK:
            for d in in_copies(k + 2, slot):
                d.start()
    for k in range(max(0, NCHUNK - 2), NCHUNK):
        for d in out_copies(k, k % 2):
            d.wait()


def _tc_body(lg_ref, bx_ref, ts_ref, s_ref, n_ref, l_ref, ob_ref):
    x = lg_ref[...]
    m = jnp.max(x, axis=0)
    iota = lax.broadcasted_iota(jnp.int32, x.shape, 0)
    lbl = jnp.min(jnp.where(x == m[None], iota, C), axis=0)
    sig = 1.0 / (1.0 + jnp.exp(-m))
    s_ref[...] = sig
    n_ref[...] = 1.0 - sig
    l_ref[...] = lbl

    tsf = ts_ref[...].astype(jnp.float32)
    hsz = tsf[:, 0:1]
    wsz = tsf[:, 1:2]
    cx = bx_ref[:, 0, :]
    cy = bx_ref[:, 1, :]
    hw = bx_ref[:, 2, :] * 0.5
    hh = bx_ref[:, 3, :] * 0.5
    ob_ref[:, 0, :] = (cx - hw) * wsz
    ob_ref[:, 1, :] = (cy - hh) * hsz
    ob_ref[:, 2, :] = (cx + hw) * wsz
    ob_ref[:, 3, :] = (cy + hh) * hsz


def kernel(pred_logits, pred_boxes, target_sizes):
    lg = jnp.transpose(pred_logits, (2, 0, 1))
    bx = jnp.transpose(pred_boxes, (0, 2, 1))
    ts = target_sizes.reshape(2 * B)

    mesh = plsc.VectorSubcoreMesh(core_axis_name="c", subcore_axis_name="s")
    sc_out_type = [
        jax.ShapeDtypeStruct((B, SC_N), jnp.float32),
        jax.ShapeDtypeStruct((B, SC_N), jnp.float32),
        jax.ShapeDtypeStruct((B, SC_N), jnp.int32),
        jax.ShapeDtypeStruct((B, 4, SC_N), jnp.float32),
    ]
    sc_scratch = [
        pltpu.VMEM((2, C, W), jnp.float32),
        pltpu.VMEM((2, 4, W), jnp.float32),
        pltpu.VMEM((2, W), jnp.float32),
        pltpu.VMEM((2, W), jnp.float32),
        pltpu.VMEM((2, W), jnp.int32),
        pltpu.VMEM((2, 4, W), jnp.float32),
        pltpu.VMEM((2 * B,), jnp.int32),
        pltpu.SemaphoreType.DMA,
        pltpu.SemaphoreType.DMA,
        pltpu.SemaphoreType.DMA,
        pltpu.SemaphoreType.DMA,
    ]
    sc_f = pl.kernel(_sc_body, out_type=sc_out_type, mesh=mesh,
                     scratch_types=sc_scratch,
                     compiler_params=pltpu.CompilerParams(
                         needs_layout_passes=False))
    sc_s, sc_n, sc_l, sc_b = sc_f(lg, bx, ts)

    nblk = (TC_N + NB - 1) // NB
    off = SC_N // NB
    tc_f = pl.pallas_call(
        _tc_body,
        grid=(nblk,),
        in_specs=[
            pl.BlockSpec((C, B, NB), lambda j: (0, 0, off + j)),
            pl.BlockSpec((B, 4, NB), lambda j: (0, 0, off + j)),
            pl.BlockSpec((B, 2), lambda j: (0, 0)),
        ],
        out_specs=[
            pl.BlockSpec((B, NB), lambda j: (0, j)),
            pl.BlockSpec((B, NB), lambda j: (0, j)),
            pl.BlockSpec((B, NB), lambda j: (0, j)),
            pl.BlockSpec((B, 4, NB), lambda j: (0, 0, j)),
        ],
        out_shape=[
            jax.ShapeDtypeStruct((B, TC_N), jnp.float32),
            jax.ShapeDtypeStruct((B, TC_N), jnp.float32),
            jax.ShapeDtypeStruct((B, TC_N), jnp.int32),
            jax.ShapeDtypeStruct((B, 4, TC_N), jnp.float32),
        ],
        compiler_params=pltpu.CompilerParams(skip_device_barrier=True),
    )
    tc_s, tc_n, tc_l, tc_b = tc_f(lg, bx, target_sizes)

    s = jnp.concatenate([sc_s, tc_s], axis=1)
    n = jnp.concatenate([sc_n, tc_n], axis=1)
    l = jnp.concatenate([sc_l, tc_l], axis=1)
    b4 = jnp.concatenate([sc_b, tc_b], axis=2)
    return s, n, l, jnp.transpose(b4, (0, 2, 1))


# speedup vs baseline: 1.0652x; 1.0051x over previous
import jax
import jax.numpy as jnp
from jax import lax
from jax.experimental import pallas as pl
from jax.experimental.pallas import tpu as pltpu
from jax.experimental.pallas import tpu_sc as plsc

B, N, C = 8, 20000, 91
L = 16
W = 512
NCHUNK = 4
RW = NCHUNK * W
SC_N = 4 * RW
TC_N = N - SC_N
NB = 2048
assert SC_N % NB == 0


def _sc_body(lg_hbm, bx_hbm, ts_hbm, o_s, o_n, o_l, o_b,
             lg_v, bx_v, s_v, n_v, l_v, ob_v, ts_v,
             isem0, isem1, osem0, osem1):
    cid = lax.axis_index("c")
    sid = lax.axis_index("s")
    wid = sid * 2 + cid
    img = lax.shift_right_logical(wid, 2)
    nbase = (wid & 3) * RW
    lane = lax.iota(jnp.int32, L)
    isem = (isem0, isem1)
    osem = (osem0, osem1)

    pltpu.sync_copy(ts_hbm, ts_v)
    swv = plsc.load_gather(ts_v, [lane * 0 + (2 * img + 1)]).astype(jnp.float32)
    shv = plsc.load_gather(ts_v, [lane * 0 + 2 * img]).astype(jnp.float32)

    def in_copies(k, slot):
        n0 = nbase + k * W
        return (
            pltpu.make_async_copy(lg_hbm.at[:, img, pl.ds(n0, W)],
                                  lg_v.at[slot], isem[slot]),
            pltpu.make_async_copy(bx_hbm.at[img, :, pl.ds(n0, W)],
                                  bx_v.at[slot], isem[slot]),
        )

    def out_copies(k, slot):
        n0 = nbase + k * W
        return (
            pltpu.make_async_copy(s_v.at[slot], o_s.at[img, pl.ds(n0, W)],
                                  osem[slot]),
            pltpu.make_async_copy(n_v.at[slot], o_n.at[img, pl.ds(n0, W)],
                                  osem[slot]),
            pltpu.make_async_copy(l_v.at[slot], o_l.at[img, pl.ds(n0, W)],
                                  osem[slot]),
            pltpu.make_async_copy(ob_v.at[slot], o_b.at[img, :, pl.ds(n0, W)],
                                  osem[slot]),
        )

    def compute(slot):
        def grp(g, c2):
            base = g * L
            dsl = pl.ds(base, L)
            m = lg_v[slot, 0, dsl]
            lbl = jnp.zeros((L,), jnp.int32)
            for c in range(1, C, 2):
                va = lg_v[slot, c, dsl]
                vb = lg_v[slot, c + 1, dsl]
                mp = jnp.maximum(va, vb)
                gt = mp > m
                m = jnp.maximum(m, mp)
                lbl = jnp.where(gt, jnp.full((L,), c, jnp.int32), lbl)
            pos = base + lane
            vwin = plsc.load_gather(lg_v, [lane * 0 + slot, lbl, pos])
            lbl = lbl + (vwin != m).astype(jnp.int32)
            sig = 1.0 / (1.0 + jnp.exp(-m))
            s_v[slot, dsl] = sig
            n_v[slot, dsl] = 1.0 - sig
            l_v[slot, dsl] = lbl

            cx = bx_v[slot, 0, dsl]
            cy = bx_v[slot, 1, dsl]
            hw = bx_v[slot, 2, dsl] * 0.5
            hh = bx_v[slot, 3, dsl] * 0.5
            ob_v[slot, 0, dsl] = (cx - hw) * swv
            ob_v[slot, 1, dsl] = (cy - hh) * shv
            ob_v[slot, 2, dsl] = (cx + hw) * swv
            ob_v[slot, 3, dsl] = (cy + hh) * shv
            return c2

        lax.fori_loop(0, W // L, grp, 0)

    for k in range(min(2, NCHUNK)):
        for d in in_copies(k, k):
            d.start()
    for k in range(NCHUNK):
        slot = k % 2
        for d in in_copies(k, slot):
            d.wait()
        if k >= 2:
            for d in out_copies(k - 2, slot):
                d.wait()
        compute(slot)
        for d in out_copies(k, slot):
            d.start()
        if k + 2 < NCHUNK:
            for d in in_copies(k + 2, slot):
                d.start()
    for k in range(max(0, NCHUNK - 2), NCHUNK):
        for d in out_copies(k, k % 2):
            d.wait()


def _tc_body(lg_ref, bx_ref, ts_ref, s_ref, n_ref, l_ref, ob_ref):
    x = lg_ref[...]
    m = jnp.max(x, axis=0)
    iota = lax.broadcasted_iota(jnp.int32, x.shape, 0)
    lbl = jnp.min(jnp.where(x == m[None], iota, C), axis=0)
    sig = 1.0 / (1.0 + jnp.exp(-m))
    s_ref[...] = sig
    n_ref[...] = 1.0 - sig
    l_ref[...] = lbl

    tsf = ts_ref[...].astype(jnp.float32)
    hsz = tsf[:, 0:1]
    wsz = tsf[:, 1:2]
    cx = bx_ref[:, 0, :]
    cy = bx_ref[:, 1, :]
    hw = bx_ref[:, 2, :] * 0.5
    hh = bx_ref[:, 3, :] * 0.5
    ob_ref[:, 0, :] = (cx - hw) * wsz
    ob_ref[:, 1, :] = (cy - hh) * hsz
    ob_ref[:, 2, :] = (cx + hw) * wsz
    ob_ref[:, 3, :] = (cy + hh) * hsz


def kernel(pred_logits, pred_boxes, target_sizes):
    lg = jnp.transpose(pred_logits, (2, 0, 1))
    bx = jnp.transpose(pred_boxes, (0, 2, 1))
    ts = target_sizes.reshape(2 * B)

    mesh = plsc.VectorSubcoreMesh(core_axis_name="c", subcore_axis_name="s")
    sc_out_type = [
        jax.ShapeDtypeStruct((B, SC_N), jnp.float32),
        jax.ShapeDtypeStruct((B, SC_N), jnp.float32),
        jax.ShapeDtypeStruct((B, SC_N), jnp.int32),
        jax.ShapeDtypeStruct((B, 4, SC_N), jnp.float32),
    ]
    sc_scratch = [
        pltpu.VMEM((2, C, W), jnp.float32),
        pltpu.VMEM((2, 4, W), jnp.float32),
        pltpu.VMEM((2, W), jnp.float32),
        pltpu.VMEM((2, W), jnp.float32),
        pltpu.VMEM((2, W), jnp.int32),
        pltpu.VMEM((2, 4, W), jnp.float32),
        pltpu.VMEM((2 * B,), jnp.int32),
        pltpu.SemaphoreType.DMA,
        pltpu.SemaphoreType.DMA,
        pltpu.SemaphoreType.DMA,
        pltpu.SemaphoreType.DMA,
    ]
    sc_f = pl.kernel(_sc_body, out_type=sc_out_type, mesh=mesh,
                     scratch_types=sc_scratch,
                     compiler_params=pltpu.CompilerParams(
                         needs_layout_passes=False))
    sc_s, sc_n, sc_l, sc_b = sc_f(lg, bx, ts)

    nblk = (TC_N + NB - 1) // NB
    off = SC_N // NB
    tc_f = pl.pallas_call(
        _tc_body,
        grid=(nblk,),
        in_specs=[
            pl.BlockSpec((C, B, NB), lambda j: (0, 0, off + j)),
            pl.BlockSpec((B, 4, NB), lambda j: (0, 0, off + j)),
            pl.BlockSpec((B, 2), lambda j: (0, 0)),
        ],
        out_specs=[
            pl.BlockSpec((B, NB), lambda j: (0, j)),
            pl.BlockSpec((B, NB), lambda j: (0, j)),
            pl.BlockSpec((B, NB), lambda j: (0, j)),
            pl.BlockSpec((B, 4, NB), lambda j: (0, 0, j)),
        ],
        out_shape=[
            jax.ShapeDtypeStruct((B, TC_N), jnp.float32),
            jax.ShapeDtypeStruct((B, TC_N), jnp.float32),
            jax.ShapeDtypeStruct((B, TC_N), jnp.int32),
            jax.ShapeDtypeStruct((B, 4, TC_N), jnp.float32),
        ],
        compiler_params=pltpu.CompilerParams(skip_device_barrier=True),
    )
    tc_s, tc_n, tc_l, tc_b = tc_f(lg, bx, target_sizes)

    s = jnp.concatenate([sc_s, tc_s], axis=1)
    n = jnp.concatenate([sc_n, tc_n], axis=1)
    l = jnp.concatenate([sc_l, tc_l], axis=1)
    b4 = jnp.concatenate([sc_b, tc_b], axis=2)
    return s, n, l, jnp.transpose(b4, (0, 2, 1))


# speedup vs baseline: 1.1539x; 1.0832x over previous
import jax
import jax.numpy as jnp
from jax import lax
from jax.experimental import pallas as pl
from jax.experimental.pallas import tpu as pltpu
from jax.experimental.pallas import tpu_sc as plsc

B, N, C = 8, 20000, 91
L = 16
W = 512
NCHUNK = 3
RW = NCHUNK * W
SC_N = 4 * RW
TC_N = N - SC_N
NB = 2048
assert SC_N % NB == 0


def _sc_body(lg_hbm, bx_hbm, ts_hbm, o_s, o_n, o_l, o_b,
             lg_v, bx_v, s_v, n_v, l_v, ob_v, ts_v,
             isem0, isem1, osem0, osem1):
    cid = lax.axis_index("c")
    sid = lax.axis_index("s")
    wid = sid * 2 + cid
    img = lax.shift_right_logical(wid, 2)
    nbase = (wid & 3) * RW
    lane = lax.iota(jnp.int32, L)
    isem = (isem0, isem1)
    osem = (osem0, osem1)

    pltpu.sync_copy(ts_hbm, ts_v)
    swv = plsc.load_gather(ts_v, [lane * 0 + (2 * img + 1)]).astype(jnp.float32)
    shv = plsc.load_gather(ts_v, [lane * 0 + 2 * img]).astype(jnp.float32)

    def in_copies(k, slot):
        n0 = nbase + k * W
        return (
            pltpu.make_async_copy(lg_hbm.at[:, img, pl.ds(n0, W)],
                                  lg_v.at[slot], isem[slot]),
            pltpu.make_async_copy(bx_hbm.at[img, :, pl.ds(n0, W)],
                                  bx_v.at[slot], isem[slot]),
        )

    def out_copies(k, slot):
        n0 = nbase + k * W
        return (
            pltpu.make_async_copy(s_v.at[slot], o_s.at[img, pl.ds(n0, W)],
                                  osem[slot]),
            pltpu.make_async_copy(n_v.at[slot], o_n.at[img, pl.ds(n0, W)],
                                  osem[slot]),
            pltpu.make_async_copy(l_v.at[slot], o_l.at[img, pl.ds(n0, W)],
                                  osem[slot]),
            pltpu.make_async_copy(ob_v.at[slot], o_b.at[img, :, pl.ds(n0, W)],
                                  osem[slot]),
        )

    def compute(slot):
        def grp(g, c2):
            base = g * L
            dsl = pl.ds(base, L)
            m = lg_v[slot, 0, dsl]
            lbl = jnp.zeros((L,), jnp.int32)
            for c in range(1, C, 2):
                va = lg_v[slot, c, dsl]
                vb = lg_v[slot, c + 1, dsl]
                mp = jnp.maximum(va, vb)
                gt = mp > m
                m = jnp.maximum(m, mp)
                lbl = jnp.where(gt, jnp.full((L,), c, jnp.int32), lbl)
            pos = base + lane
            vwin = plsc.load_gather(lg_v, [lane * 0 + slot, lbl, pos])
            lbl = lbl + (vwin != m).astype(jnp.int32)
            sig = 1.0 / (1.0 + jnp.exp(-m))
            s_v[slot, dsl] = sig
            n_v[slot, dsl] = 1.0 - sig
            l_v[slot, dsl] = lbl

            cx = bx_v[slot, 0, dsl]
            cy = bx_v[slot, 1, dsl]
            hw = bx_v[slot, 2, dsl] * 0.5
            hh = bx_v[slot, 3, dsl] * 0.5
            ob_v[slot, 0, dsl] = (cx - hw) * swv
            ob_v[slot, 1, dsl] = (cy - hh) * shv
            ob_v[slot, 2, dsl] = (cx + hw) * swv
            ob_v[slot, 3, dsl] = (cy + hh) * shv
            return c2

        lax.fori_loop(0, W // L, grp, 0)

    for k in range(min(2, NCHUNK)):
        for d in in_copies(k, k):
            d.start()
    for k in range(NCHUNK):
        slot = k % 2
        for d in in_copies(k, slot):
            d.wait()
        if k >= 2:
            for d in out_copies(k - 2, slot):
                d.wait()
        compute(slot)
        for d in out_copies(k, slot):
            d.start()
        if k + 2 < NCHUNK:
            for d in in_copies(k + 2, slot):
                d.start()
    for k in range(max(0, NCHUNK - 2), NCHUNK):
        for d in out_copies(k, k % 2):
            d.wait()


def _tc_body(lg_ref, bx_ref, ts_ref, s_ref, n_ref, l_ref, ob_ref):
    x = lg_ref[...]
    m = jnp.max(x, axis=0)
    iota = lax.broadcasted_iota(jnp.int32, x.shape, 0)
    lbl = jnp.min(jnp.where(x == m[None], iota, C), axis=0)
    sig = 1.0 / (1.0 + jnp.exp(-m))
    s_ref[...] = sig
    n_ref[...] = 1.0 - sig
    l_ref[...] = lbl

    tsf = ts_ref[...].astype(jnp.float32)
    hsz = tsf[:, 0:1]
    wsz = tsf[:, 1:2]
    cx = bx_ref[:, 0, :]
    cy = bx_ref[:, 1, :]
    hw = bx_ref[:, 2, :] * 0.5
    hh = bx_ref[:, 3, :] * 0.5
    ob_ref[:, 0, :] = (cx - hw) * wsz
    ob_ref[:, 1, :] = (cy - hh) * hsz
    ob_ref[:, 2, :] = (cx + hw) * wsz
    ob_ref[:, 3, :] = (cy + hh) * hsz


def kernel(pred_logits, pred_boxes, target_sizes):
    lg = jnp.transpose(pred_logits, (2, 0, 1))
    bx = jnp.transpose(pred_boxes, (0, 2, 1))
    ts = target_sizes.reshape(2 * B)

    mesh = plsc.VectorSubcoreMesh(core_axis_name="c", subcore_axis_name="s")
    sc_out_type = [
        jax.ShapeDtypeStruct((B, SC_N), jnp.float32),
        jax.ShapeDtypeStruct((B, SC_N), jnp.float32),
        jax.ShapeDtypeStruct((B, SC_N), jnp.int32),
        jax.ShapeDtypeStruct((B, 4, SC_N), jnp.float32),
    ]
    sc_scratch = [
        pltpu.VMEM((2, C, W), jnp.float32),
        pltpu.VMEM((2, 4, W), jnp.float32),
        pltpu.VMEM((2, W), jnp.float32),
        pltpu.VMEM((2, W), jnp.float32),
        pltpu.VMEM((2, W), jnp.int32),
        pltpu.VMEM((2, 4, W), jnp.float32),
        pltpu.VMEM((2 * B,), jnp.int32),
        pltpu.SemaphoreType.DMA,
        pltpu.SemaphoreType.DMA,
        pltpu.SemaphoreType.DMA,
        pltpu.SemaphoreType.DMA,
    ]
    sc_f = pl.kernel(_sc_body, out_type=sc_out_type, mesh=mesh,
                     scratch_types=sc_scratch,
                     compiler_params=pltpu.CompilerParams(
                         needs_layout_passes=False))
    sc_s, sc_n, sc_l, sc_b = sc_f(lg, bx, ts)

    nblk = (TC_N + NB - 1) // NB
    off = SC_N // NB
    tc_f = pl.pallas_call(
        _tc_body,
        grid=(nblk,),
        in_specs=[
            pl.BlockSpec((C, B, NB), lambda j: (0, 0, off + j)),
            pl.BlockSpec((B, 4, NB), lambda j: (0, 0, off + j)),
            pl.BlockSpec((B, 2), lambda j: (0, 0)),
        ],
        out_specs=[
            pl.BlockSpec((B, NB), lambda j: (0, j)),
            pl.BlockSpec((B, NB), lambda j: (0, j)),
            pl.BlockSpec((B, NB), lambda j: (0, j)),
            pl.BlockSpec((B, 4, NB), lambda j: (0, 0, j)),
        ],
        out_shape=[
            jax.ShapeDtypeStruct((B, TC_N), jnp.float32),
            jax.ShapeDtypeStruct((B, TC_N), jnp.float32),
            jax.ShapeDtypeStruct((B, TC_N), jnp.int32),
            jax.ShapeDtypeStruct((B, 4, TC_N), jnp.float32),
        ],
        compiler_params=pltpu.CompilerParams(skip_device_barrier=True),
    )
    tc_s, tc_n, tc_l, tc_b = tc_f(lg, bx, target_sizes)

    s = jnp.concatenate([sc_s, tc_s], axis=1)
    n = jnp.concatenate([sc_n, tc_n], axis=1)
    l = jnp.concatenate([sc_l, tc_l], axis=1)
    b4 = jnp.concatenate([sc_b, tc_b], axis=2)
    return s, n, l, jnp.transpose(b4, (0, 2, 1))
